# baseline (speedup 1.0000x reference)
import jax
import jax.numpy as jnp
from jax.experimental import pallas as pl
from jax.experimental.pallas import tpu as pltpu
import math

_BINS = 10
_ALPHA = 0.75
_ROWS = 65536
_COLS = 80
_N = float(_ROWS * _COLS)
_BLOCK_ROWS = 16384
_CH = 128
_L2E = 1.4426950408889634
_LN2 = 0.6931471805599453
_XTHRESH = [math.log((b / 9.9999) / (1.0 - b / 9.9999)) for b in range(1, _BINS)]


def _fold(v):
    w = v.reshape(_CH // 8, 8, _COLS)
    out = w[0]
    for k in range(1, _CH // 8):
        out = out + w[k]
    return out


def _ghm_kernel(x_ref, t_ref, out_ref, acc_ref):
    i = pl.program_id(0)
    nblk = pl.num_programs(0)

    zero = jnp.zeros((8, _COLS), jnp.float32)

    def body(j, carry):
        utot, cnts, sums = carry
        base = j * _CH
        x = x_ref[pl.ds(base, _CH), :]
        t = t_ref[pl.ds(base, _CH), :]
        xb = jax.lax.bitcast_convert_type(x, jnp.uint32)
        tb = jax.lax.bitcast_convert_type(t, jnp.uint32)
        xs = jax.lax.bitcast_convert_type(xb ^ (tb << jnp.uint32(8)),
                                          jnp.float32)
        p = jnp.exp2(xs * _L2E)
        u = jnp.log2(1.0 + p)
        utot = utot + _fold(u)
        new_c = []
        new_s = []
        for k in range(_BINS - 1):
            mask = xs >= _XTHRESH[k]
            new_c.append(cnts[k] + _fold(jnp.where(mask, 1.0, 0.0)))
            new_s.append(sums[k] + _fold(jnp.where(mask, u, 0.0)))
        return utot, tuple(new_c), tuple(new_s)

    init = (zero, (zero,) * (_BINS - 1), (zero,) * (_BINS - 1))
    utot, cnts, sums = jax.lax.fori_loop(0, _BLOCK_ROWS // _CH, body, init)

    flat = jnp.stack(list(cnts) + list(sums) + [utot], axis=0)

    @pl.when(i == 0)
    def _init():
        acc_ref[...] = flat

    @pl.when(i > 0)
    def _accum():
        acc_ref[...] += flat

    @pl.when(i == nblk - 1)
    def _epilogue():
        a = jnp.sum(jnp.sum(acc_ref[...], axis=2), axis=1, keepdims=True)
        cge = a[0:_BINS - 1]
        sge = a[_BINS - 1:2 * _BINS - 2]
        ut = a[2 * _BINS - 2:2 * _BINS - 1]
        counts = jnp.concatenate(
            [_N - cge[0:1], cge[:-1] - cge[1:], cge[-1:]], axis=0)
        usums = jnp.concatenate(
            [ut - sge[0:1], sge[:-1] - sge[1:], sge[-1:]], axis=0)
        bsums = _LN2 * usums
        acc_sum = (1.0 - _ALPHA) * counts
        w = jnp.where(counts >= 1.0, _N / jnp.maximum(acc_sum, 1e-12), 0.0)
        nonempty = jnp.sum((counts >= 1.0).astype(jnp.float32))
        w = w / jnp.maximum(nonempty, 1.0)
        w = jnp.maximum(w, 0.0001)
        total = jnp.sum(w * bsums, axis=0, keepdims=True)
        out_ref[...] = total / _N


def kernel(x, target):
    grid = (_ROWS // _BLOCK_ROWS,)
    out = pl.pallas_call(
        _ghm_kernel,
        grid=grid,
        in_specs=[
            pl.BlockSpec((_BLOCK_ROWS, _COLS), lambda i: (i, 0)),
            pl.BlockSpec((_BLOCK_ROWS, _COLS), lambda i: (i, 0)),
        ],
        out_specs=pl.BlockSpec((1, 1), lambda i: (0, 0)),
        out_shape=jax.ShapeDtypeStruct((1, 1), jnp.float32),
        scratch_shapes=[pltpu.VMEM((2 * _BINS - 1, 8, _COLS), jnp.float32)],
    )(x, target)
    return out[0, 0]


# speedup vs baseline: 1.0254x; 1.0254x over previous
import jax
import jax.numpy as jnp
from jax.experimental import pallas as pl
from jax.experimental.pallas import tpu as pltpu
import math

_BINS = 10
_ALPHA = 0.75
_ROWS = 65536
_COLS = 80
_N = float(_ROWS * _COLS)
_BLOCK_ROWS = 4096
_CH = 128
_L2E = 1.4426950408889634
_LN2 = 0.6931471805599453
_XTHRESH = [math.log((b / 9.9999) / (1.0 - b / 9.9999)) for b in range(1, _BINS)]


def _fold(v):
    w = v.reshape(_CH // 8, 8, _COLS)
    out = w[0]
    for k in range(1, _CH // 8):
        out = out + w[k]
    return out


def _ghm_kernel(x_ref, t_ref, out_ref, acc_ref):
    i = pl.program_id(0)
    nblk = pl.num_programs(0)

    zero = jnp.zeros((8, _COLS), jnp.float32)

    def body(j, carry):
        utot, cnts, sums = carry
        base = j * _CH
        x = x_ref[pl.ds(base, _CH), :]
        t = t_ref[pl.ds(base, _CH), :]
        xb = jax.lax.bitcast_convert_type(x, jnp.uint32)
        tb = jax.lax.bitcast_convert_type(t, jnp.uint32)
        xs = jax.lax.bitcast_convert_type(xb ^ (tb << jnp.uint32(8)),
                                          jnp.float32)
        p = jnp.exp2(xs * _L2E)
        u = jnp.log2(1.0 + p)
        utot = utot + _fold(u)
        new_c = []
        new_s = []
        for k in range(_BINS - 1):
            mask = xs >= _XTHRESH[k]
            new_c.append(cnts[k] + _fold(jnp.where(mask, 1.0, 0.0)))
            new_s.append(sums[k] + _fold(jnp.where(mask, u, 0.0)))
        return utot, tuple(new_c), tuple(new_s)

    init = (zero, (zero,) * (_BINS - 1), (zero,) * (_BINS - 1))
    utot, cnts, sums = jax.lax.fori_loop(0, _BLOCK_ROWS // _CH, body, init)

    flat = jnp.stack(list(cnts) + list(sums) + [utot], axis=0)

    @pl.when(i == 0)
    def _init():
        acc_ref[...] = flat

    @pl.when(i > 0)
    def _accum():
        acc_ref[...] += flat

    @pl.when(i == nblk - 1)
    def _epilogue():
        a = jnp.sum(jnp.sum(acc_ref[...], axis=2), axis=1, keepdims=True)
        cge = a[0:_BINS - 1]
        sge = a[_BINS - 1:2 * _BINS - 2]
        ut = a[2 * _BINS - 2:2 * _BINS - 1]
        counts = jnp.concatenate(
            [_N - cge[0:1], cge[:-1] - cge[1:], cge[-1:]], axis=0)
        usums = jnp.concatenate(
            [ut - sge[0:1], sge[:-1] - sge[1:], sge[-1:]], axis=0)
        bsums = _LN2 * usums
        acc_sum = (1.0 - _ALPHA) * counts
        w = jnp.where(counts >= 1.0, _N / jnp.maximum(acc_sum, 1e-12), 0.0)
        nonempty = jnp.sum((counts >= 1.0).astype(jnp.float32))
        w = w / jnp.maximum(nonempty, 1.0)
        w = jnp.maximum(w, 0.0001)
        total = jnp.sum(w * bsums, axis=0, keepdims=True)
        out_ref[...] = total / _N


def kernel(x, target):
    grid = (_ROWS // _BLOCK_ROWS,)
    out = pl.pallas_call(
        _ghm_kernel,
        grid=grid,
        in_specs=[
            pl.BlockSpec((_BLOCK_ROWS, _COLS), lambda i: (i, 0)),
            pl.BlockSpec((_BLOCK_ROWS, _COLS), lambda i: (i, 0)),
        ],
        out_specs=pl.BlockSpec((1, 1), lambda i: (0, 0)),
        out_shape=jax.ShapeDtypeStruct((1, 1), jnp.float32),
        scratch_shapes=[pltpu.VMEM((2 * _BINS - 1, 8, _COLS), jnp.float32)],
    )(x, target)
    return out[0, 0]


# speedup vs baseline: 1.0256x; 1.0002x over previous
import jax
import jax.numpy as jnp
from jax.experimental import pallas as pl
from jax.experimental.pallas import tpu as pltpu
import math

_BINS = 10
_ALPHA = 0.75
_ROWS = 65536
_COLS = 80
_N = float(_ROWS * _COLS)
_BLOCK_ROWS = 2048
_CH = 128
_L2E = 1.4426950408889634
_LN2 = 0.6931471805599453
_XTHRESH = [math.log((b / 9.9999) / (1.0 - b / 9.9999)) for b in range(1, _BINS)]


def _fold(v):
    w = v.reshape(_CH // 8, 8, _COLS)
    out = w[0]
    for k in range(1, _CH // 8):
        out = out + w[k]
    return out


def _ghm_kernel(x_ref, t_ref, out_ref, acc_ref):
    i = pl.program_id(0)
    nblk = pl.num_programs(0)

    zero = jnp.zeros((8, _COLS), jnp.float32)

    def body(j, carry):
        utot, cnts, sums = carry
        base = j * _CH
        x = x_ref[pl.ds(base, _CH), :]
        t = t_ref[pl.ds(base, _CH), :]
        xb = jax.lax.bitcast_convert_type(x, jnp.uint32)
        tb = jax.lax.bitcast_convert_type(t, jnp.uint32)
        xs = jax.lax.bitcast_convert_type(xb ^ (tb << jnp.uint32(8)),
                                          jnp.float32)
        p = jnp.exp2(xs * _L2E)
        u = jnp.log2(1.0 + p)
        utot = utot + _fold(u)
        new_c = []
        new_s = []
        for k in range(_BINS - 1):
            mask = xs >= _XTHRESH[k]
            new_c.append(cnts[k] + _fold(jnp.where(mask, 1.0, 0.0)))
            new_s.append(sums[k] + _fold(jnp.where(mask, u, 0.0)))
        return utot, tuple(new_c), tuple(new_s)

    init = (zero, (zero,) * (_BINS - 1), (zero,) * (_BINS - 1))
    utot, cnts, sums = jax.lax.fori_loop(0, _BLOCK_ROWS // _CH, body, init)

    flat = jnp.stack(list(cnts) + list(sums) + [utot], axis=0)

    @pl.when(i == 0)
    def _init():
        acc_ref[...] = flat

    @pl.when(i > 0)
    def _accum():
        acc_ref[...] += flat

    @pl.when(i == nblk - 1)
    def _epilogue():
        a = jnp.sum(jnp.sum(acc_ref[...], axis=2), axis=1, keepdims=True)
        cge = a[0:_BINS - 1]
        sge = a[_BINS - 1:2 * _BINS - 2]
        ut = a[2 * _BINS - 2:2 * _BINS - 1]
        counts = jnp.concatenate(
            [_N - cge[0:1], cge[:-1] - cge[1:], cge[-1:]], axis=0)
        usums = jnp.concatenate(
            [ut - sge[0:1], sge[:-1] - sge[1:], sge[-1:]], axis=0)
        bsums = _LN2 * usums
        acc_sum = (1.0 - _ALPHA) * counts
        w = jnp.where(counts >= 1.0, _N / jnp.maximum(acc_sum, 1e-12), 0.0)
        nonempty = jnp.sum((counts >= 1.0).astype(jnp.float32))
        w = w / jnp.maximum(nonempty, 1.0)
        w = jnp.maximum(w, 0.0001)
        total = jnp.sum(w * bsums, axis=0, keepdims=True)
        out_ref[...] = total / _N


def kernel(x, target):
    grid = (_ROWS // _BLOCK_ROWS,)
    out = pl.pallas_call(
        _ghm_kernel,
        grid=grid,
        in_specs=[
            pl.BlockSpec((_BLOCK_ROWS, _COLS), lambda i: (i, 0)),
            pl.BlockSpec((_BLOCK_ROWS, _COLS), lambda i: (i, 0)),
        ],
        out_specs=pl.BlockSpec((1, 1), lambda i: (0, 0)),
        out_shape=jax.ShapeDtypeStruct((1, 1), jnp.float32),
        scratch_shapes=[pltpu.VMEM((2 * _BINS - 1, 8, _COLS), jnp.float32)],
    )(x, target)
    return out[0, 0]
